# Initial kernel scaffold; baseline (speedup 1.0000x reference)
import jax
import jax.numpy as jnp
from jax.experimental import pallas as pl


def kernel(x, codebooks):
    raise NotImplementedError("write your pallas kernel here")


# speedup vs baseline: 1.4848x; 1.4848x over previous
import functools

import jax
import jax.numpy as jnp
from jax.experimental import pallas as pl

N_Q_ = 8
K_ = 2048
D_ = 64
B_ = 8
T_ = 2048
TILE_ = 512


def _rvq_kernel(x_ref, cb_ref, cbt_ref, qout_ref, idx_ref, loss_ref):
    step = pl.program_id(0)

    @pl.when(step == 0)
    def _init():
        loss_ref[...] = jnp.zeros_like(loss_ref)

    residual = x_ref[0].T
    qout_acc = jnp.zeros((TILE_, D_), dtype=jnp.float32)
    loss_acc = loss_ref[...]
    lane_iota = jax.lax.broadcasted_iota(jnp.int32, (TILE_, K_), 1)

    for q in range(N_Q_):
        cbt = cbt_ref[q]
        m = jnp.dot(residual, cbt, preferred_element_type=jnp.float32)
        ssx = jnp.sum(residual * residual, axis=1, keepdims=True)
        ssc = jnp.sum(cbt * cbt, axis=0, keepdims=True)
        d = (ssx - 2.0 * m) + ssc
        dmin = jnp.min(d, axis=1, keepdims=True)
        idx = jnp.min(
            jnp.where(d == dmin, lane_iota, K_), axis=1
        )
        idx_ref[0, q, :] = idx
        onehot = (lane_iota == idx[:, None]).astype(jnp.float32)
        quantized = jnp.dot(onehot, cb_ref[q], preferred_element_type=jnp.float32)
        e = quantized - residual
        q_out = residual + e
        qout_acc = qout_acc + q_out
        loss_acc = loss_acc + jnp.sum(e * e, axis=(0, 1), keepdims=True) * (
            1.0 / (B_ * D_ * T_)
        )
        residual = residual - q_out

    qout_ref[0] = qout_acc.T
    loss_ref[...] = loss_acc


@functools.partial(jax.jit, static_argnames=())
def kernel(x, codebooks):
    cbt = jnp.transpose(codebooks, (0, 2, 1))
    n_tiles = T_ // TILE_
    grid = (B_ * n_tiles,)

    def x_map(i):
        return (i // n_tiles, 0, i % n_tiles)

    qout, idx, loss = pl.pallas_call(
        _rvq_kernel,
        grid=grid,
        in_specs=[
            pl.BlockSpec((1, D_, TILE_), x_map),
            pl.BlockSpec((N_Q_, K_, D_), lambda i: (0, 0, 0)),
            pl.BlockSpec((N_Q_, D_, K_), lambda i: (0, 0, 0)),
        ],
        out_specs=[
            pl.BlockSpec((1, D_, TILE_), x_map),
            pl.BlockSpec((1, N_Q_, TILE_), x_map),
            pl.BlockSpec((1, 1), lambda i: (0, 0)),
        ],
        out_shape=[
            jax.ShapeDtypeStruct((B_, D_, T_), jnp.float32),
            jax.ShapeDtypeStruct((B_, N_Q_, T_), jnp.int32),
            jax.ShapeDtypeStruct((1, 1), jnp.float32),
        ],
    )(x, codebooks, cbt)
    return qout, idx, loss.reshape(())
